# baseline (speedup 1.0000x reference)
import jax
import jax.numpy as jnp
from jax.experimental import pallas as pl

_NUM_TYPES = 128
_ROWS_PER_BLOCK = 4096


def _onehot_block(x_ref, out_ref):
    x = x_ref[...]
    classes = jax.lax.broadcasted_iota(jnp.int32, (1, _NUM_TYPES), 1)
    out_ref[...] = (x == classes).astype(jnp.int32)


def kernel(x):
    n, s = x.shape
    flat = n * s
    xf = x.reshape(flat, 1)
    r = _ROWS_PER_BLOCK
    out = pl.pallas_call(
        _onehot_block,
        grid=(flat // r,),
        in_specs=[pl.BlockSpec((r, 1), lambda i: (i, 0))],
        out_specs=pl.BlockSpec((r, _NUM_TYPES), lambda i: (i, 0)),
        out_shape=jax.ShapeDtypeStruct((flat, _NUM_TYPES), jnp.int32),
    )(xf)
    return out.reshape(n, s, _NUM_TYPES)


# speedup vs baseline: 1.6128x; 1.6128x over previous
import jax
import jax.numpy as jnp
from jax.experimental import pallas as pl

_NUM_TYPES = 128
_ROWS_PER_BLOCK = 256


def _onehot_block(x_ref, out_ref):
    x = x_ref[...]
    classes = jax.lax.broadcasted_iota(jnp.int32, (1, _NUM_TYPES), 1)
    for j in range(x.shape[1]):
        col = x[:, j:j + 1]
        out_ref[:, j * _NUM_TYPES:(j + 1) * _NUM_TYPES] = (
            (col == classes).astype(jnp.int32))


def kernel(x):
    n, s = x.shape
    r = _ROWS_PER_BLOCK
    out = pl.pallas_call(
        _onehot_block,
        grid=(n // r,),
        in_specs=[pl.BlockSpec((r, s), lambda i: (i, 0))],
        out_specs=pl.BlockSpec((r, s * _NUM_TYPES), lambda i: (i, 0)),
        out_shape=jax.ShapeDtypeStruct((n, s * _NUM_TYPES), jnp.int32),
    )(x)
    return out.reshape(n, s, _NUM_TYPES)
